# baseline (speedup 1.0000x reference)
import jax
import jax.numpy as jnp
from jax.experimental import pallas as pl
from jax.experimental.pallas import tpu as pltpu

_NB = 32


def _conv_stage_kernel(x_ref, bc_ref, b1_ref, w2_ref, b2_ref, out_ref):
    nb = x_ref.shape[0]
    x3 = x_ref[...]
    xcat = jnp.concatenate(
        [x3[:, 0:30, :], x3[:, 1:31, :], x3[:, 2:32, :]],
        axis=-1).reshape(nb * 30, 96)
    y1 = jnp.dot(xcat, bc_ref[...], preferred_element_type=jnp.float32)
    y1 = y1.reshape(nb, 30, 30, 64) + b1_ref[0]
    t = y1.reshape(nb, 15, 2, 15, 2, 64)
    p1 = jnp.maximum(jnp.maximum(t[:, :, 0, :, 0], t[:, :, 0, :, 1]),
                     jnp.maximum(t[:, :, 1, :, 0], t[:, :, 1, :, 1]))
    p1 = p1.astype(jnp.bfloat16)
    acc = jnp.broadcast_to(b2_ref[0], (nb * 169, 192)).astype(jnp.float32)
    for i in range(3):
        cat = jnp.concatenate(
            [p1[:, i:i + 13, j:j + 13, :] for j in range(3)],
            axis=-1).reshape(nb * 169, 192)
        acc = acc + jnp.dot(cat, w2_ref[i], preferred_element_type=jnp.float32)
    t2 = acc.reshape(nb, 13, 13, 192)[:, :12, :12, :]
    t2 = t2.reshape(nb, 6, 2, 6, 2, 192)
    p2 = jnp.maximum(jnp.maximum(t2[:, :, 0, :, 0], t2[:, :, 0, :, 1]),
                     jnp.maximum(t2[:, :, 1, :, 0], t2[:, :, 1, :, 1]))
    out_ref[...] = p2.reshape(nb, 6912).astype(jnp.bfloat16)


def _conv_stage(x3, bcat, b1, w2r, b2):
    n = x3.shape[0]
    return pl.pallas_call(
        _conv_stage_kernel,
        out_shape=jax.ShapeDtypeStruct((n, 6912), jnp.bfloat16),
        grid_spec=pltpu.PrefetchScalarGridSpec(
            num_scalar_prefetch=0,
            grid=(n // _NB,),
            in_specs=[
                pl.BlockSpec((_NB, 32, 32), lambda i: (i, 0, 0)),
                pl.BlockSpec((96, 1920), lambda i: (0, 0)),
                pl.BlockSpec((1, 64), lambda i: (0, 0)),
                pl.BlockSpec((3, 192, 192), lambda i: (0, 0, 0)),
                pl.BlockSpec((1, 192), lambda i: (0, 0)),
            ],
            out_specs=pl.BlockSpec((_NB, 6912), lambda i: (i, 0)),
        ),
        compiler_params=pltpu.CompilerParams(
            dimension_semantics=("parallel",),
            vmem_limit_bytes=56 * 1024 * 1024),
    )(x3, bcat, b1, w2r, b2)


def _fc_stage_kernel(x_ref, w1_ref, b1_ref, w2_ref, b2_ref,
                     h2_ref, p_ref, h_acc):
    k = pl.program_id(1)

    @pl.when(k == 0)
    def _():
        h_acc[...] = jnp.broadcast_to(b1_ref[...], h_acc.shape)

    h_acc[...] += jnp.dot(x_ref[...], w1_ref[...],
                          preferred_element_type=jnp.float32)

    @pl.when(k == pl.num_programs(1) - 1)
    def _():
        h = h_acc[...]
        halves = (h[:, :512], h[:, 512:])
        h2_ref[...] = halves[1]
        for head in range(2):
            logits = jnp.dot(halves[head], w2_ref[head],
                             preferred_element_type=jnp.float32) + b2_ref[head]
            m = jnp.max(logits, axis=-1, keepdims=True)
            e = jnp.exp(logits - m)
            p_ref[head] = e / jnp.sum(e, axis=-1, keepdims=True)


def _fc_stage(flat, w1, b1, w2, b2):
    m, k = flat.shape
    tm = min(512, m)
    tk = 1152
    cpad = w2.shape[-1]
    return pl.pallas_call(
        _fc_stage_kernel,
        out_shape=(jax.ShapeDtypeStruct((m, 512), jnp.float32),
                   jax.ShapeDtypeStruct((2, m, cpad), jnp.float32)),
        grid_spec=pltpu.PrefetchScalarGridSpec(
            num_scalar_prefetch=0,
            grid=(m // tm, k // tk),
            in_specs=[
                pl.BlockSpec((tm, tk), lambda i, kk: (i, kk)),
                pl.BlockSpec((tk, 1024), lambda i, kk: (kk, 0)),
                pl.BlockSpec((1, 1024), lambda i, kk: (0, 0)),
                pl.BlockSpec((2, 512, cpad), lambda i, kk: (0, 0, 0)),
                pl.BlockSpec((2, 1, cpad), lambda i, kk: (0, 0, 0)),
            ],
            out_specs=[
                pl.BlockSpec((tm, 512), lambda i, kk: (i, 0)),
                pl.BlockSpec((2, tm, cpad), lambda i, kk: (0, i, 0)),
            ],
            scratch_shapes=[pltpu.VMEM((tm, 1024), jnp.float32)],
        ),
        compiler_params=pltpu.CompilerParams(
            dimension_semantics=("parallel", "arbitrary"),
            vmem_limit_bytes=48 * 1024 * 1024),
    )(flat, w1, b1, w2, b2)


def _build_banded_conv1(conv1_wm):
    w1r = conv1_wm.reshape(3, 3, 64)
    b4 = jnp.zeros((3, 32, 30, 64), conv1_wm.dtype)
    wo = jnp.arange(30)
    for j in range(3):
        b4 = b4.at[:, wo + j, wo, :].set(w1r[:, j, :][:, None, :])
    return b4.reshape(96, 1920)


def kernel(x_nchw, conv1_wm, conv1_b, conv2_wm, conv2_b,
           fc1_w, fc1_b, fc2_w, fc2_b):
    n = x_nchw.shape[0]
    x3 = x_nchw.reshape(n, 32, 32).astype(jnp.bfloat16)
    bcat = _build_banded_conv1(conv1_wm).astype(jnp.bfloat16)
    flat = _conv_stage(x3, bcat, conv1_b.reshape(1, 64),
                       conv2_wm.reshape(3, 192, 192).astype(jnp.bfloat16),
                       conv2_b.reshape(1, 192))
    h2, probs = _fc_stage(flat, fc1_w.astype(jnp.bfloat16), fc1_b, fc2_w, fc2_b)
    return h2, probs[0, :, :20], probs[1, :, :100]


# speedup vs baseline: 2.4556x; 2.4556x over previous
import jax
import jax.numpy as jnp
from jax.experimental import pallas as pl
from jax.experimental.pallas import tpu as pltpu

_NB = 32


def _conv_stage_kernel(x_ref, bc_ref, w2_ref, b2_ref, out_ref):
    nb = x_ref.shape[1]
    xt = x_ref[...]
    xcat = jnp.concatenate(
        [xt[0:30], xt[1:31], xt[2:32], jnp.ones((30, nb, 1), xt.dtype)],
        axis=-1).reshape(30 * nb, 97)
    y1 = jnp.dot(xcat, bc_ref[...], preferred_element_type=jnp.float32)
    y4 = y1.reshape(15, 2, nb, 1920)
    u = jnp.maximum(y4[:, 0], y4[:, 1])
    t = u.reshape(15, nb, 15, 128)
    p1 = jnp.maximum(t[:, :, :, :64], t[:, :, :, 64:])
    acc = jnp.broadcast_to(b2_ref[0], (13 * nb * 13, 192)).astype(jnp.float32)
    for i in range(3):
        cat = jnp.concatenate(
            [p1[i:i + 13, :, j:j + 13, :] for j in range(3)],
            axis=-1).reshape(13 * nb * 13, 192)
        acc = acc + jnp.dot(cat, w2_ref[i], preferred_element_type=jnp.float32)
    a5 = acc.reshape(13, nb, 13, 192)[:12].reshape(6, 2, nb, 13, 192)
    hp = jnp.maximum(a5[:, 0], a5[:, 1])
    w5 = hp[:, :, :12, :].reshape(6, nb, 6, 2, 192)
    p2 = jnp.maximum(w5[:, :, :, 0, :], w5[:, :, :, 1, :])
    out_ref[...] = p2.reshape(6, nb, 1152)


def _conv_stage(xt, bcat, w2r, b2):
    n = xt.shape[1]
    return pl.pallas_call(
        _conv_stage_kernel,
        out_shape=jax.ShapeDtypeStruct((6, n, 1152), jnp.float32),
        grid_spec=pltpu.PrefetchScalarGridSpec(
            num_scalar_prefetch=0,
            grid=(n // _NB,),
            in_specs=[
                pl.BlockSpec((32, _NB, 32), lambda i: (0, i, 0)),
                pl.BlockSpec((97, 1920), lambda i: (0, 0)),
                pl.BlockSpec((3, 192, 192), lambda i: (0, 0, 0)),
                pl.BlockSpec((1, 192), lambda i: (0, 0)),
            ],
            out_specs=pl.BlockSpec((6, _NB, 1152), lambda i: (0, i, 0)),
        ),
        compiler_params=pltpu.CompilerParams(
            dimension_semantics=("parallel",),
            vmem_limit_bytes=56 * 1024 * 1024),
    )(xt, bcat, w2r, b2)


def _fc_stage_kernel(x_ref, w1_ref, b1_ref, w2_ref, b2_ref,
                     h2_ref, p_ref, h_acc):
    k = pl.program_id(1)

    @pl.when(k == 0)
    def _():
        h_acc[...] = jnp.broadcast_to(b1_ref[...], h_acc.shape)

    h_acc[...] += jnp.dot(x_ref[0], w1_ref[0],
                          preferred_element_type=jnp.float32)

    @pl.when(k == pl.num_programs(1) - 1)
    def _():
        h = h_acc[...]
        halves = (h[:, :512], h[:, 512:])
        h2_ref[...] = halves[1]
        for head in range(2):
            logits = jnp.dot(halves[head], w2_ref[head],
                             preferred_element_type=jnp.float32) + b2_ref[head]
            m = jnp.max(logits, axis=-1, keepdims=True)
            e = jnp.exp(logits - m)
            p_ref[head] = e / jnp.sum(e, axis=-1, keepdims=True)


def _fc_stage(flat6, w16, b1, w2, b2):
    _, m, tk = flat6.shape
    tm = min(512, m)
    cpad = w2.shape[-1]
    return pl.pallas_call(
        _fc_stage_kernel,
        out_shape=(jax.ShapeDtypeStruct((m, 512), jnp.float32),
                   jax.ShapeDtypeStruct((2, m, cpad), jnp.float32)),
        grid_spec=pltpu.PrefetchScalarGridSpec(
            num_scalar_prefetch=0,
            grid=(m // tm, 6),
            in_specs=[
                pl.BlockSpec((1, tm, tk), lambda i, kk: (kk, i, 0)),
                pl.BlockSpec((1, tk, 1024), lambda i, kk: (kk, 0, 0)),
                pl.BlockSpec((1, 1024), lambda i, kk: (0, 0)),
                pl.BlockSpec((2, 512, cpad), lambda i, kk: (0, 0, 0)),
                pl.BlockSpec((2, 1, cpad), lambda i, kk: (0, 0, 0)),
            ],
            out_specs=[
                pl.BlockSpec((tm, 512), lambda i, kk: (i, 0)),
                pl.BlockSpec((2, tm, cpad), lambda i, kk: (0, i, 0)),
            ],
            scratch_shapes=[pltpu.VMEM((tm, 1024), jnp.float32)],
        ),
        compiler_params=pltpu.CompilerParams(
            dimension_semantics=("parallel", "arbitrary"),
            vmem_limit_bytes=48 * 1024 * 1024),
    )(flat6, w16, b1, w2, b2)


def _build_banded_conv1(conv1_wm, conv1_b):
    w1r = conv1_wm.reshape(3, 3, 64)
    b4 = jnp.zeros((3, 32, 30, 64), conv1_wm.dtype)
    wo = jnp.arange(30)
    for j in range(3):
        b4 = b4.at[:, wo + j, wo, :].set(w1r[:, j, :][:, None, :])
    bias_row = jnp.tile(conv1_b, 30).reshape(1, 1920)
    return jnp.concatenate([b4.reshape(96, 1920), bias_row], axis=0)


def kernel(x_nchw, conv1_wm, conv1_b, conv2_wm, conv2_b,
           fc1_w, fc1_b, fc2_w, fc2_b):
    n = x_nchw.shape[0]
    xt = x_nchw.reshape(n, 32, 32).transpose(1, 0, 2)
    bcat = _build_banded_conv1(conv1_wm, conv1_b)
    flat6 = _conv_stage(xt, bcat, conv2_wm.reshape(3, 192, 192),
                        conv2_b.reshape(1, 192))
    w16 = fc1_w.reshape(6, 1152, 1024)
    h2, probs = _fc_stage(flat6, w16, fc1_b, fc2_w, fc2_b)
    return h2, probs[0, :, :20], probs[1, :, :100]
